# Initial kernel scaffold; baseline (speedup 1.0000x reference)
import jax
import jax.numpy as jnp
from jax.experimental import pallas as pl


def kernel(node_features, W_in, b_in, W0, al0, ar0, g0, be0, W1, al1, ar1, g1, be1, Wp1, bp1, Wp2, bp2, edge_index):
    raise NotImplementedError("write your pallas kernel here")


# speedup vs baseline: 4.5172x; 4.5172x over previous
import functools

import jax
import jax.numpy as jnp
from jax import lax
from jax.experimental import pallas as pl


def _dotT(a, b):
    return lax.dot_general(a, b, (((0,), (0,)), ((), ())),
                           preferred_element_type=jnp.float32)


def _gat_block(hT, W_ref, Al_ref, Ar_ref, g_ref, be_ref, bias_ref,
               N, H, DH, bB):
    D_h = hT.shape[0]
    hwT = _dotT(W_ref[...], hT)
    elT = _dotT(Al_ref[...], hwT)
    erT = _dotT(Ar_ref[...], hwT)
    el = jnp.concatenate([elT[:, s * bB:(s + 1) * bB] for s in range(N)],
                         axis=0)
    outs = []
    for d in range(N):
        er_d = erT[:, d * bB:(d + 1) * bB]
        e = el + jnp.tile(er_d, (N, 1))
        e = jnp.where(e >= 0, e, 0.2 * e)
        e = e + bias_ref[:, d:d + 1]
        m = e[0:H]
        for s in range(1, N):
            m = jnp.maximum(m, e[s * H:(s + 1) * H])
        ex = jnp.exp(e - jnp.tile(m, (N, 1)))
        den = ex[0:H]
        for s in range(1, N):
            den = den + ex[s * H:(s + 1) * H]
        r = 1.0 / den
        acc = None
        for s in range(N):
            a4 = ex[s * H:(s + 1) * H] * r
            a128 = jnp.concatenate(
                [jnp.broadcast_to(a4[h:h + 1, :], (DH, bB)) for h in range(H)],
                axis=0)
            term = a128 * hwT[:, s * bB:(s + 1) * bB]
            acc = term if acc is None else acc + term
        outs.append(acc)
    oT = jnp.concatenate(outs, axis=1)
    y = oT + hT
    mu = jnp.mean(y, axis=0, keepdims=True)
    yc = y - mu
    var = jnp.mean(yc * yc, axis=0, keepdims=True)
    yn = yc * lax.rsqrt(var + 1e-5) * g_ref[...] + be_ref[...]
    return jnp.where(yn > 0, yn, jnp.exp(jnp.minimum(yn, 0.0)) - 1.0)


def _fused_kernel(x_ref, Win_ref, bin_ref,
                  W0_ref, Al0_ref, Ar0_ref, g0_ref, be0_ref,
                  W1_ref, Al1_ref, Ar1_ref, g1_ref, be1_ref,
                  Wp1_ref, bp1_ref, Wp2_ref, bp2_ref, bias_ref,
                  ge_ref, ne_ref, *, N, H, DH):
    bB = x_ref.shape[0]
    Win = Win_ref[...]
    hTs = []
    for n in range(N):
        hn = jnp.dot(x_ref[:, n, :], Win,
                     preferred_element_type=jnp.float32)
        hTs.append(hn.T)
    hT = jnp.concatenate(hTs, axis=1) + bin_ref[...]

    h1 = _gat_block(hT, W0_ref, Al0_ref, Ar0_ref, g0_ref, be0_ref,
                    bias_ref, N, H, DH, bB)
    h2 = _gat_block(h1, W1_ref, Al1_ref, Ar1_ref, g1_ref, be1_ref,
                    bias_ref, N, H, DH, bB)

    for n in range(N):
        ne_ref[:, n, :] = h2[:, n * bB:(n + 1) * bB].T

    pool = h2[:, 0:bB]
    for n in range(1, N):
        pool = pool + h2[:, n * bB:(n + 1) * bB]
    pool = pool * (1.0 / N)
    z = _dotT(Wp1_ref[...], pool) + bp1_ref[...]
    z = jnp.maximum(z, 0.0)
    gT = _dotT(Wp2_ref[...], z) + bp2_ref[...]
    ge_ref[...] = gT.T


def kernel(node_features, W_in, b_in, W0, al0, ar0, g0, be0,
           W1, al1, ar1, g1, be1, Wp1, bp1, Wp2, bp2, edge_index):
    B, N, D_in = node_features.shape
    D_h = W_in.shape[1]
    H, DH = al0.shape
    f32 = jnp.float32

    bB = 512
    while B % bB:
        bB //= 2

    def expand(a):
        out = jnp.zeros((D_h, H), f32)
        rows = jnp.arange(DH)[:, None] + DH * jnp.arange(H)[None, :]
        return out.at[rows, jnp.arange(H)[None, :]].set(a.T)

    Al0, Ar0 = expand(al0), expand(ar0)
    Al1, Ar1 = expand(al1), expand(ar1)

    src, dst = edge_index[0], edge_index[1]
    biasM = jnp.full((N, N), -1e30, f32).at[dst, src].set(0.0)
    bias_cols = jnp.repeat(biasM.T, H, axis=0)

    col = lambda v: v.reshape(-1, 1).astype(f32)

    grid = (B // bB,)
    full = lambda s: pl.BlockSpec(s, lambda i: (0,) * len(s))
    out_shape = (
        jax.ShapeDtypeStruct((B, D_h), f32),
        jax.ShapeDtypeStruct((B, N, D_h), f32),
    )
    fn = functools.partial(_fused_kernel, N=N, H=H, DH=DH)
    ge, ne = pl.pallas_call(
        fn,
        grid=grid,
        in_specs=[
            pl.BlockSpec((bB, N, D_in), lambda i: (i, 0, 0)),
            full((D_in, D_h)), full((D_h, 1)),
            full((D_h, D_h)), full((D_h, H)), full((D_h, H)),
            full((D_h, 1)), full((D_h, 1)),
            full((D_h, D_h)), full((D_h, H)), full((D_h, H)),
            full((D_h, 1)), full((D_h, 1)),
            full((D_h, D_h)), full((D_h, 1)),
            full((D_h, D_h)), full((D_h, 1)),
            full((N * H, N)),
        ],
        out_specs=(
            pl.BlockSpec((bB, D_h), lambda i: (i, 0)),
            pl.BlockSpec((bB, N, D_h), lambda i: (i, 0, 0)),
        ),
        out_shape=out_shape,
    )(node_features.astype(f32), W_in, col(b_in),
      W0, Al0, Ar0, col(g0), col(be0),
      W1, Al1, Ar1, col(g1), col(be1),
      Wp1, col(bp1), Wp2, col(bp2), bias_cols)
    return ge, ne


# speedup vs baseline: 5.3159x; 1.1768x over previous
import functools

import jax
import jax.numpy as jnp
from jax import lax
from jax.experimental import pallas as pl


def _dotT(a, b):
    return lax.dot_general(a, b, (((0,), (0,)), ((), ())),
                           preferred_element_type=jnp.float32)


def _gat_block(hT, W_ref, Al_ref, Ar_ref, g_ref, be_ref, bias_ref,
               N, H, DH, bB):
    D_h = hT.shape[0]
    hwT = _dotT(W_ref[...], hT)
    elT = _dotT(Al_ref[...], hwT)
    erT = _dotT(Ar_ref[...], hwT)
    el = jnp.concatenate([elT[:, s * bB:(s + 1) * bB] for s in range(N)],
                         axis=0)
    outs = []
    for d in range(N):
        er_d = erT[:, d * bB:(d + 1) * bB]
        e = el + jnp.tile(er_d, (N, 1))
        e = jnp.where(e >= 0, e, 0.2 * e)
        e = e + bias_ref[:, d:d + 1]
        m = e[0:H]
        for s in range(1, N):
            m = jnp.maximum(m, e[s * H:(s + 1) * H])
        ex = jnp.exp(e - jnp.tile(m, (N, 1)))
        den = ex[0:H]
        for s in range(1, N):
            den = den + ex[s * H:(s + 1) * H]
        r = 1.0 / den
        slabs = []
        for h in range(H):
            acc = None
            for s in range(N):
                a1 = ex[s * H + h:s * H + h + 1] * r[h:h + 1]
                term = a1 * hwT[h * DH:(h + 1) * DH, s * bB:(s + 1) * bB]
                acc = term if acc is None else acc + term
            slabs.append(acc)
        outs.append(jnp.concatenate(slabs, axis=0))
    oT = jnp.concatenate(outs, axis=1)
    y = oT + hT
    ones = jnp.full((D_h, 1), 1.0 / D_h, jnp.float32)
    mu = _dotT(ones, y)
    yc = y - mu
    var = _dotT(ones, yc * yc)
    yn = yc * lax.rsqrt(var + 1e-5) * g_ref[...] + be_ref[...]
    return jnp.where(yn > 0, yn, jnp.exp(jnp.minimum(yn, 0.0)) - 1.0)


def _fused_kernel(x_ref, Win_ref, bin_ref,
                  W0_ref, Al0_ref, Ar0_ref, g0_ref, be0_ref,
                  W1_ref, Al1_ref, Ar1_ref, g1_ref, be1_ref,
                  Wp1_ref, bp1_ref, Wp2_ref, bp2_ref, bias_ref,
                  ge_ref, ne_ref, *, N, H, DH):
    bB = x_ref.shape[0]
    Win = Win_ref[...]
    hTs = []
    for n in range(N):
        hn = jnp.dot(x_ref[:, n, :], Win,
                     preferred_element_type=jnp.float32)
        hTs.append(hn.T)
    hT = jnp.concatenate(hTs, axis=1) + bin_ref[...]

    h1 = _gat_block(hT, W0_ref, Al0_ref, Ar0_ref, g0_ref, be0_ref,
                    bias_ref, N, H, DH, bB)
    h2 = _gat_block(h1, W1_ref, Al1_ref, Ar1_ref, g1_ref, be1_ref,
                    bias_ref, N, H, DH, bB)

    for n in range(N):
        ne_ref[:, n, :] = h2[:, n * bB:(n + 1) * bB].T

    pool = h2[:, 0:bB]
    for n in range(1, N):
        pool = pool + h2[:, n * bB:(n + 1) * bB]
    pool = pool * (1.0 / N)
    z = _dotT(Wp1_ref[...], pool) + bp1_ref[...]
    z = jnp.maximum(z, 0.0)
    gT = _dotT(Wp2_ref[...], z) + bp2_ref[...]
    ge_ref[...] = gT.T


def kernel(node_features, W_in, b_in, W0, al0, ar0, g0, be0,
           W1, al1, ar1, g1, be1, Wp1, bp1, Wp2, bp2, edge_index):
    B, N, D_in = node_features.shape
    D_h = W_in.shape[1]
    H, DH = al0.shape
    f32 = jnp.float32

    bB = 512
    while B % bB:
        bB //= 2

    blk_mask = (jnp.arange(D_h)[:, None] // DH
                == jnp.arange(H)[None, :]).astype(f32)

    def expand(a):
        return jnp.tile(a.T.astype(f32), (H, 1)) * blk_mask

    Al0, Ar0 = expand(al0), expand(ar0)
    Al1, Ar1 = expand(al1), expand(ar1)

    src, dst = edge_index[0], edge_index[1]
    eq_d = (dst[:, None] == jnp.arange(N)[None, :]).astype(f32)
    eq_s = (src[:, None] == jnp.arange(N)[None, :]).astype(f32)
    adj = eq_d.T @ eq_s
    biasM = jnp.where(adj > 0, 0.0, -1e30)
    bias_cols = jnp.repeat(biasM.T, H, axis=0)

    col = lambda v: v.reshape(-1, 1).astype(f32)

    grid = (B // bB,)
    full = lambda s: pl.BlockSpec(s, lambda i: (0,) * len(s))
    out_shape = (
        jax.ShapeDtypeStruct((B, D_h), f32),
        jax.ShapeDtypeStruct((B, N, D_h), f32),
    )
    fn = functools.partial(_fused_kernel, N=N, H=H, DH=DH)
    ge, ne = pl.pallas_call(
        fn,
        grid=grid,
        in_specs=[
            pl.BlockSpec((bB, N, D_in), lambda i: (i, 0, 0)),
            full((D_in, D_h)), full((D_h, 1)),
            full((D_h, D_h)), full((D_h, H)), full((D_h, H)),
            full((D_h, 1)), full((D_h, 1)),
            full((D_h, D_h)), full((D_h, H)), full((D_h, H)),
            full((D_h, 1)), full((D_h, 1)),
            full((D_h, D_h)), full((D_h, 1)),
            full((D_h, D_h)), full((D_h, 1)),
            full((N * H, N)),
        ],
        out_specs=(
            pl.BlockSpec((bB, D_h), lambda i: (i, 0)),
            pl.BlockSpec((bB, N, D_h), lambda i: (i, 0, 0)),
        ),
        out_shape=out_shape,
    )(node_features.astype(f32), W_in, col(b_in),
      W0, Al0, Ar0, col(g0), col(be0),
      W1, Al1, Ar1, col(g1), col(be1),
      Wp1, col(bp1), Wp2, col(bp2), bias_cols)
    return ge, ne


# speedup vs baseline: 5.7055x; 1.0733x over previous
import functools

import jax
import jax.numpy as jnp
from jax import lax
from jax.experimental import pallas as pl


def _dotT(a, b):
    return lax.dot_general(a, b, (((0,), (0,)), ((), ())),
                           preferred_element_type=jnp.float32)


def _expand_attn(a_ref, N, H, DH):
    D_h = H * DH
    tiled = jnp.tile(a_ref[...].T, (H, 1))
    row_blk = lax.broadcasted_iota(jnp.int32, (D_h, H), 0) // DH
    col = lax.broadcasted_iota(jnp.int32, (D_h, H), 1)
    return jnp.where(row_blk == col, tiled, 0.0)


def _col(row_ref):
    return row_ref[...].T


def _gat_block(hT, W_ref, Al, Ar, g_ref, be_ref, bias, N, H, DH, bB):
    D_h = hT.shape[0]
    hwT = _dotT(W_ref[...], hT)
    elT = _dotT(Al, hwT)
    erT = _dotT(Ar, hwT)
    el = jnp.concatenate([elT[:, s * bB:(s + 1) * bB] for s in range(N)],
                         axis=0)
    outs = []
    for d in range(N):
        er_d = erT[:, d * bB:(d + 1) * bB]
        e = el + jnp.tile(er_d, (N, 1))
        e = jnp.where(e >= 0, e, 0.2 * e)
        e = e + bias[:, d:d + 1]
        m = e[0:H]
        for s in range(1, N):
            m = jnp.maximum(m, e[s * H:(s + 1) * H])
        ex = jnp.exp(e - jnp.tile(m, (N, 1)))
        den = ex[0:H]
        for s in range(1, N):
            den = den + ex[s * H:(s + 1) * H]
        r = 1.0 / den
        slabs = []
        for h in range(H):
            acc = None
            for s in range(N):
                a1 = ex[s * H + h:s * H + h + 1] * r[h:h + 1]
                term = a1 * hwT[h * DH:(h + 1) * DH, s * bB:(s + 1) * bB]
                acc = term if acc is None else acc + term
            slabs.append(acc)
        outs.append(jnp.concatenate(slabs, axis=0))
    oT = jnp.concatenate(outs, axis=1)
    y = oT + hT
    ones = jnp.full((D_h, 1), 1.0 / D_h, jnp.float32)
    mu = _dotT(ones, y)
    yc = y - mu
    var = _dotT(ones, yc * yc)
    yn = yc * lax.rsqrt(var + 1e-5) * _col(g_ref) + _col(be_ref)
    return jnp.where(yn > 0, yn, jnp.exp(jnp.minimum(yn, 0.0)) - 1.0)


def _fused_kernel(x_ref, Win_ref, bin_ref,
                  W0_ref, al0_ref, ar0_ref, g0_ref, be0_ref,
                  W1_ref, al1_ref, ar1_ref, g1_ref, be1_ref,
                  Wp1_ref, bp1_ref, Wp2_ref, bp2_ref, edge_ref,
                  ge_ref, ne_ref, *, N, H, DH):
    bB = x_ref.shape[0]
    D_h = H * DH

    edge = edge_ref[...]
    enc = (edge[1:2, :] * N + edge[0:1, :]).astype(jnp.float32)
    s_of_row = lax.broadcasted_iota(jnp.int32, (N * H, N), 0) // H
    d_of_col = lax.broadcasted_iota(jnp.int32, (N * H, N), 1)
    P = (d_of_col * N + s_of_row).astype(jnp.float32)
    hit = jnp.zeros((N * H, N), jnp.float32)
    for e in range(edge.shape[1]):
        v = enc[0:1, e:e + 1]
        hit = jnp.maximum(hit, jnp.where(P == v, 1.0, 0.0))
    bias = (hit - 1.0) * 1e30

    Al0 = _expand_attn(al0_ref, N, H, DH)
    Ar0 = _expand_attn(ar0_ref, N, H, DH)
    Al1 = _expand_attn(al1_ref, N, H, DH)
    Ar1 = _expand_attn(ar1_ref, N, H, DH)

    Win = Win_ref[...]
    b_in = _col(bin_ref)
    hTs = []
    for n in range(N):
        hn = jnp.dot(x_ref[:, n, :], Win,
                     preferred_element_type=jnp.float32)
        hTs.append(hn.T)
    hT = jnp.concatenate(hTs, axis=1) + b_in

    h1 = _gat_block(hT, W0_ref, Al0, Ar0, g0_ref, be0_ref,
                    bias, N, H, DH, bB)
    h2 = _gat_block(h1, W1_ref, Al1, Ar1, g1_ref, be1_ref,
                    bias, N, H, DH, bB)

    for n in range(N):
        ne_ref[:, n, :] = h2[:, n * bB:(n + 1) * bB].T

    pool = h2[:, 0:bB]
    for n in range(1, N):
        pool = pool + h2[:, n * bB:(n + 1) * bB]
    pool = pool * (1.0 / N)
    z = _dotT(Wp1_ref[...], pool) + _col(bp1_ref)
    z = jnp.maximum(z, 0.0)
    gT = _dotT(Wp2_ref[...], z) + _col(bp2_ref)
    ge_ref[...] = gT.T


def kernel(node_features, W_in, b_in, W0, al0, ar0, g0, be0,
           W1, al1, ar1, g1, be1, Wp1, bp1, Wp2, bp2, edge_index):
    B, N, D_in = node_features.shape
    D_h = W_in.shape[1]
    H, DH = al0.shape
    E = edge_index.shape[1]
    f32 = jnp.float32

    bB = 512
    while B % bB:
        bB //= 2

    row = lambda v: v.reshape(1, -1)

    grid = (B // bB,)
    full = lambda s: pl.BlockSpec(s, lambda i: (0,) * len(s))
    out_shape = (
        jax.ShapeDtypeStruct((B, D_h), f32),
        jax.ShapeDtypeStruct((B, N, D_h), f32),
    )
    fn = functools.partial(_fused_kernel, N=N, H=H, DH=DH)
    ge, ne = pl.pallas_call(
        fn,
        grid=grid,
        in_specs=[
            pl.BlockSpec((bB, N, D_in), lambda i: (i, 0, 0)),
            full((D_in, D_h)), full((1, D_h)),
            full((D_h, D_h)), full((H, DH)), full((H, DH)),
            full((1, D_h)), full((1, D_h)),
            full((D_h, D_h)), full((H, DH)), full((H, DH)),
            full((1, D_h)), full((1, D_h)),
            full((D_h, D_h)), full((1, D_h)),
            full((D_h, D_h)), full((1, D_h)),
            full((2, E)),
        ],
        out_specs=(
            pl.BlockSpec((bB, D_h), lambda i: (i, 0)),
            pl.BlockSpec((bB, N, D_h), lambda i: (i, 0, 0)),
        ),
        out_shape=out_shape,
    )(node_features, W_in, row(b_in),
      W0, al0, ar0, row(g0), row(be0),
      W1, al1, ar1, row(g1), row(be1),
      Wp1, row(bp1), Wp2, row(bp2), edge_index)
    return ge, ne
